# baseline (speedup 1.0000x reference)
import functools

import jax
import jax.numpy as jnp
from jax import lax
from jax.experimental import pallas as pl
from jax.experimental.pallas import tpu as pltpu
from jax.experimental.pallas import tpu_sc as plsc


def _sc_gather_rows(knn_flat, current_node, n1):
    b = current_node.shape[0]
    k = knn_flat.shape[1]
    info = plsc.get_sparse_core_info()
    nw = info.num_cores * info.num_subcores
    bpw = b // nw
    mesh = plsc.VectorSubcoreMesh(core_axis_name="c", subcore_axis_name="s")

    @functools.partial(
        pl.kernel,
        mesh=mesh,
        compiler_params=pltpu.CompilerParams(use_tc_tiling_on_sc=False),
        out_type=jax.ShapeDtypeStruct((b, k), jnp.int32),
        scratch_types=[
            pltpu.VMEM((bpw,), jnp.int32),
            pltpu.VMEM((bpw, k), jnp.int32),
            pltpu.SemaphoreType.DMA,
        ],
    )
    def gather_kernel(cn_hbm, knn_hbm, out_hbm, idx_v, rows_v, sem):
        wid = lax.axis_index("s") * info.num_cores + lax.axis_index("c")
        base = wid * bpw
        pltpu.sync_copy(cn_hbm.at[pl.ds(base, bpw)], idx_v)
        for i in range(bpw // 16):
            v = idx_v[pl.ds(i * 16, 16)]
            row = (base + i * 16 + lax.iota(jnp.int32, 16)) * n1 + v
            idx_v[pl.ds(i * 16, 16)] = row
        pltpu.async_copy(knn_hbm.at[idx_v], rows_v, sem).wait()
        pltpu.sync_copy(rows_v, out_hbm.at[pl.ds(base, bpw)])

    return gather_kernel(current_node, knn_flat)


def _tc_body(psi_x_ref, psi_y_ref, cx_ref, cy_ref, vis_ref, dem_ref,
             knn_ref, cn_ref, cap_ref, used_ref, p_ref,
             lp_ref, mask_ref):
    bb, n1 = dem_ref.shape
    k = knn_ref.shape[1]

    vis = vis_ref[...]
    dem = dem_ref[...]
    cap = cap_ref[...]
    used = used_ref[...]
    rem = cap - used

    col = lax.broadcasted_iota(jnp.int32, (bb, n1), 1)
    m0 = vis | (dem > rem)
    cn = cn_ref[...]
    at_depot = cn == 0
    has_cust = jnp.any((~m0) & (col > 0), axis=1, keepdims=True)
    is0 = col == 0
    mask = (is0 & at_depot & has_cust) | ((~is0) & m0)

    cx = cx_ref[...]
    cy = cy_ref[...]
    sel = col == cn
    zero = jnp.float32(0.0)
    ccx = jnp.sum(jnp.where(sel, cx, zero), axis=1, keepdims=True)
    ccy = jnp.sum(jnp.where(sel, cy, zero), axis=1, keepdims=True)

    vis_frac = jnp.sum(vis.astype(jnp.float32), axis=1, keepdims=True) / jnp.float32(n1)
    lam = p_ref[0]
    mu = p_ref[1]
    step_frac = p_ref[2]
    q0 = (ccx * p_ref[5] + ccy * p_ref[7] + rem * p_ref[9] + used * p_ref[11]
          + step_frac * p_ref[13] + vis_frac * p_ref[15] + p_ref[3])
    q1 = (ccx * p_ref[6] + ccy * p_ref[8] + rem * p_ref[10] + used * p_ref[12]
          + step_frac * p_ref[14] + vis_frac * p_ref[16] + p_ref[4])

    amp = psi_x_ref[...] * q0 + psi_y_ref[...] * q1
    dx = cx - ccx
    dy = cy - ccy
    dist = jnp.sqrt(dx * dx + dy * dy)
    scores = lam * amp - mu * dist

    knn = knn_ref[...]
    kmask = col == 0
    for j in range(k):
        kmask = kmask | (col == knn[:, j:j + 1])

    neg = jnp.float32(-1e9)
    ms = jnp.where(mask | (~kmask), neg, scores)
    mx = jnp.max(ms, axis=1, keepdims=True)
    shifted = ms - mx
    e = jnp.exp(shifted)
    lp_ref[...] = shifted - jnp.log(jnp.sum(e, axis=1, keepdims=True))
    mask_ref[...] = mask


def kernel(psi_prime, knn_indices, coords, visited, demands, capacity,
           used_capacity, current_node, step, n_customers, W_ctx, b_ctx, lam, mu):
    b, n1, _ = psi_prime.shape
    k = knn_indices.shape[-1]

    knn_flat = knn_indices.reshape(b * n1, k)
    knn_cur = _sc_gather_rows(knn_flat, current_node.astype(jnp.int32), n1)

    psi_x = psi_prime[:, :, 0]
    psi_y = psi_prime[:, :, 1]
    cx = coords[:, :, 0]
    cy = coords[:, :, 1]

    step_f = jnp.asarray(step).astype(jnp.float32)
    denom_f = jnp.maximum(jnp.asarray(n_customers), 1).astype(jnp.float32)
    step_frac = step_f / denom_f
    params = jnp.concatenate([
        jnp.stack([lam.astype(jnp.float32), mu.astype(jnp.float32), step_frac,
                   b_ctx[0], b_ctx[1]]),
        W_ctx.reshape(-1),
    ])

    bb = 64
    grid = (b // bb,)
    row = lambda i: (i, 0)
    log_probs, mask = pl.pallas_call(
        _tc_body,
        grid=grid,
        in_specs=[
            pl.BlockSpec((bb, n1), row),
            pl.BlockSpec((bb, n1), row),
            pl.BlockSpec((bb, n1), row),
            pl.BlockSpec((bb, n1), row),
            pl.BlockSpec((bb, n1), row),
            pl.BlockSpec((bb, n1), row),
            pl.BlockSpec((bb, k), row),
            pl.BlockSpec((bb, 1), row),
            pl.BlockSpec((bb, 1), row),
            pl.BlockSpec((bb, 1), row),
            pl.BlockSpec(memory_space=pltpu.SMEM),
        ],
        out_specs=[
            pl.BlockSpec((bb, n1), row),
            pl.BlockSpec((bb, n1), row),
        ],
        out_shape=[
            jax.ShapeDtypeStruct((b, n1), jnp.float32),
            jax.ShapeDtypeStruct((b, n1), jnp.bool_),
        ],
    )(psi_x, psi_y, cx, cy, visited, demands, knn_cur,
      current_node.reshape(b, 1).astype(jnp.int32),
      capacity.reshape(b, 1), used_capacity.reshape(b, 1), params)
    return (log_probs, mask)


# speedup vs baseline: 1.6530x; 1.6530x over previous
import jax
import jax.numpy as jnp
from jax import lax
from jax.experimental import pallas as pl
from jax.experimental.pallas import tpu as pltpu


def _tc_body(knn_hbm, cn_smem, psix_ref, psiy_ref, cx_ref, cy_ref,
             vis_ref, dem_ref, cn_ref, cap_ref, used_ref, p_ref,
             lp_ref, mask_ref,
             knn_v, sem):
    bb, n1 = dem_ref.shape
    k = knn_v.shape[1]
    row0 = pl.program_id(0) * bb

    copies = []
    for j in range(bb):
        cn_j = cn_smem[row0 + j]
        cp = pltpu.make_async_copy(knn_hbm.at[row0 + j, cn_j], knn_v.at[j], sem)
        cp.start()
        copies.append(cp)

    vis = vis_ref[...]
    dem = dem_ref[...]
    cap = cap_ref[...]
    used = used_ref[...]
    rem = cap - used

    col = lax.broadcasted_iota(jnp.int32, (bb, n1), 1)
    m0 = vis | (dem > rem)
    cn = cn_ref[...]
    at_depot = cn == 0
    has_cust = jnp.any((~m0) & (col > 0), axis=1, keepdims=True)
    is0 = col == 0
    mask = (is0 & at_depot & has_cust) | ((~is0) & m0)

    cx = cx_ref[...]
    cy = cy_ref[...]
    sel = col == cn
    zero = jnp.float32(0.0)
    ccx = jnp.sum(jnp.where(sel, cx, zero), axis=1, keepdims=True)
    ccy = jnp.sum(jnp.where(sel, cy, zero), axis=1, keepdims=True)

    vis_frac = jnp.sum(vis.astype(jnp.float32), axis=1, keepdims=True) / jnp.float32(n1)
    lam = p_ref[0]
    mu = p_ref[1]
    step_frac = p_ref[2]
    q0 = (ccx * p_ref[5] + ccy * p_ref[7] + rem * p_ref[9] + used * p_ref[11]
          + step_frac * p_ref[13] + vis_frac * p_ref[15] + p_ref[3])
    q1 = (ccx * p_ref[6] + ccy * p_ref[8] + rem * p_ref[10] + used * p_ref[12]
          + step_frac * p_ref[14] + vis_frac * p_ref[16] + p_ref[4])

    amp = psix_ref[...] * q0 + psiy_ref[...] * q1
    dx = cx - ccx
    dy = cy - ccy
    dist = jnp.sqrt(dx * dx + dy * dy)
    scores = lam * amp - mu * dist

    for cp in copies:
        cp.wait()
    knn = knn_v[...]
    kmask = is0
    for j in range(k):
        kmask = kmask | (col == knn[:, j:j + 1])

    neg = jnp.float32(-1e9)
    ms = jnp.where(mask | (~kmask), neg, scores)
    mx = jnp.max(ms, axis=1, keepdims=True)
    shifted = ms - mx
    e = jnp.exp(shifted)
    lp_ref[...] = shifted - jnp.log(jnp.sum(e, axis=1, keepdims=True))
    mask_ref[...] = mask


def kernel(psi_prime, knn_indices, coords, visited, demands, capacity,
           used_capacity, current_node, step, n_customers, W_ctx, b_ctx, lam, mu):
    b, n1, _ = psi_prime.shape
    k = knn_indices.shape[-1]

    psi_x = psi_prime[:, :, 0]
    psi_y = psi_prime[:, :, 1]
    cx = coords[:, :, 0]
    cy = coords[:, :, 1]

    step_f = jnp.asarray(step).astype(jnp.float32)
    denom_f = jnp.maximum(jnp.asarray(n_customers), 1).astype(jnp.float32)
    step_frac = step_f / denom_f
    params = jnp.concatenate([
        jnp.stack([lam.astype(jnp.float32), mu.astype(jnp.float32), step_frac,
                   b_ctx[0], b_ctx[1]]),
        W_ctx.reshape(-1),
    ])

    cn_i32 = current_node.astype(jnp.int32)

    bb = 64
    grid = (b // bb,)
    row = lambda i: (i, 0)
    log_probs, mask = pl.pallas_call(
        _tc_body,
        grid=grid,
        in_specs=[
            pl.BlockSpec(memory_space=pl.ANY),
            pl.BlockSpec(memory_space=pltpu.SMEM),
            pl.BlockSpec((bb, n1), row),
            pl.BlockSpec((bb, n1), row),
            pl.BlockSpec((bb, n1), row),
            pl.BlockSpec((bb, n1), row),
            pl.BlockSpec((bb, n1), row),
            pl.BlockSpec((bb, n1), row),
            pl.BlockSpec((bb, 1), row),
            pl.BlockSpec((bb, 1), row),
            pl.BlockSpec((bb, 1), row),
            pl.BlockSpec(memory_space=pltpu.SMEM),
        ],
        out_specs=[
            pl.BlockSpec((bb, n1), row),
            pl.BlockSpec((bb, n1), row),
        ],
        out_shape=[
            jax.ShapeDtypeStruct((b, n1), jnp.float32),
            jax.ShapeDtypeStruct((b, n1), jnp.bool_),
        ],
        scratch_shapes=[
            pltpu.VMEM((bb, k), jnp.int32),
            pltpu.SemaphoreType.DMA,
        ],
    )(knn_indices, cn_i32, psi_x, psi_y, cx, cy, visited, demands,
      cn_i32.reshape(b, 1), capacity.reshape(b, 1), used_capacity.reshape(b, 1),
      params)
    return (log_probs, mask)
